# baseline (speedup 1.0000x reference)
import functools

import jax
import jax.numpy as jnp
from jax import lax
from jax.experimental import pallas as pl
from jax.experimental.pallas import tpu as pltpu
from jax.experimental.pallas import tpu_sc as plsc

_ROWS, _NPTS = 48, 4096
_CHUNKS = _NPTS // 16


def _make_sc_kernel():
    info = plsc.get_sparse_core_info()
    nc, ns = info.num_cores, info.num_subcores
    nw = nc * ns
    rows_per_w = (_ROWS + nw - 1) // nw
    mesh = plsc.VectorSubcoreMesh(core_axis_name="c", subcore_axis_name="s")

    @functools.partial(
        pl.kernel,
        mesh=mesh,
        out_type=jax.ShapeDtypeStruct((_ROWS, _NPTS), jnp.float32),
        scratch_types=[
            pltpu.VMEM((_NPTS,), jnp.float32),
            pltpu.VMEM((_NPTS,), jnp.float32),
        ],
    )
    def sc_voxel(x_hbm, o_hbm, xv, ov):
        wid = lax.axis_index("s") * nc + lax.axis_index("c")
        for k in range(rows_per_w):
            row = wid + k * nw

            @pl.when(row < _ROWS)
            def _():
                pltpu.sync_copy(x_hbm.at[row], xv)

                def red_body(i, carry):
                    mn, mx = carry
                    v = xv[pl.ds(i * 16, 16)]
                    return jnp.minimum(mn, v), jnp.maximum(mx, v)

                mn16, mx16 = lax.fori_loop(
                    0, _CHUNKS, red_body,
                    (jnp.full((16,), jnp.inf, jnp.float32),
                     jnp.full((16,), -jnp.inf, jnp.float32)))
                dn = lax.GatherDimensionNumbers(
                    offset_dims=(), collapsed_slice_dims=(0,),
                    start_index_map=(0,))
                gather = functools.partial(
                    lax.gather, dimension_numbers=dn, slice_sizes=(1,),
                    mode=lax.GatherScatterMode.PROMISE_IN_BOUNDS)
                idx = lax.iota(jnp.int32, 16)
                mn_b, mx_b = mn16, mx16
                for shift in (8, 4, 2, 1):
                    perm = (idx ^ shift)[:, None]
                    mn_b = jnp.minimum(mn_b, gather(mn_b, perm))
                    mx_b = jnp.maximum(mx_b, gather(mx_b, perm))
                bw_b = (mx_b - mn_b) / 40.0

                def emit_body(i, carry):
                    v = xv[pl.ds(i * 16, 16)]
                    q = (v - mn_b) / bw_b
                    ov[pl.ds(i * 16, 16)] = q.astype(jnp.int32).astype(jnp.float32)
                    return carry

                lax.fori_loop(0, _CHUNKS, emit_body, 0)
                pltpu.sync_copy(ov, o_hbm.at[row])

    return sc_voxel


_sc_voxel = _make_sc_kernel()


def kernel(point_cloud):
    b, n, c = point_cloud.shape
    xt = jnp.transpose(point_cloud, (2, 0, 1)).reshape(_ROWS, _NPTS)
    out = _sc_voxel(xt)
    return jnp.transpose(out.reshape(c, b, n), (1, 2, 0))


# speedup vs baseline: 10.8129x; 10.8129x over previous
import jax
import jax.numpy as jnp
from jax.experimental import pallas as pl
from jax.experimental.pallas import tpu as pltpu


def _voxel_body(x_hbm, o_hbm, x_v, o_v, in_sem, out_sem):
    pltpu.make_async_copy(x_hbm, x_v, in_sem).start()
    pltpu.make_async_copy(x_hbm, x_v, in_sem).wait()
    x = x_v[...]
    mn = jnp.min(x, axis=2, keepdims=True)
    mx = jnp.max(x, axis=2, keepdims=True)
    bw = (mx - mn) / 40.0
    o_v[...] = jnp.floor((x - mn) / bw)
    pltpu.make_async_copy(o_v, o_hbm, out_sem).start()
    pltpu.make_async_copy(o_v, o_hbm, out_sem).wait()


def kernel(point_cloud):
    b, n, c = point_cloud.shape
    xt = jnp.transpose(point_cloud, (2, 0, 1))
    out = pl.pallas_call(
        _voxel_body,
        in_specs=[pl.BlockSpec(memory_space=pltpu.MemorySpace.HBM)],
        out_specs=pl.BlockSpec(memory_space=pltpu.MemorySpace.HBM),
        out_shape=jax.ShapeDtypeStruct((c, b, n), jnp.float32),
        scratch_shapes=[
            pltpu.VMEM((c, b, n), jnp.float32),
            pltpu.VMEM((c, b, n), jnp.float32),
            pltpu.SemaphoreType.DMA,
            pltpu.SemaphoreType.DMA,
        ],
    )(xt)
    return jnp.transpose(out, (1, 2, 0))


# speedup vs baseline: 10.8806x; 1.0063x over previous
import jax
import jax.numpy as jnp
from jax.experimental import pallas as pl


def _voxel_body(x_ref, o_ref):
    x = x_ref[...]
    mn = jnp.min(x, axis=2, keepdims=True)
    mx = jnp.max(x, axis=2, keepdims=True)
    bw = (mx - mn) / 40.0
    o_ref[...] = jnp.floor((x - mn) / bw)


def kernel(point_cloud):
    b, n, c = point_cloud.shape
    xt = jnp.transpose(point_cloud, (2, 0, 1))
    out = pl.pallas_call(
        _voxel_body,
        out_shape=jax.ShapeDtypeStruct((c, b, n), jnp.float32),
    )(xt)
    return jnp.transpose(out, (1, 2, 0))
